# baseline (speedup 1.0000x reference)
import jax
import jax.numpy as jnp
from jax.experimental import pallas as pl

S_BLK = 256
B_BLK = 4


def _pe_kernel(x_ref, mask_ref, pe_ref, out_ref):
    m = mask_ref[:, 0, 0, :]
    out_ref[...] = (x_ref[...] + pe_ref[...]) * m[:, :, None]


def kernel(x, mask, pos_emb):
    bs, sl, d = x.shape
    grid = (sl // S_BLK, bs // B_BLK)
    mask4 = mask.reshape(bs, sl // S_BLK, 1, S_BLK)
    return pl.pallas_call(
        _pe_kernel,
        grid=grid,
        in_specs=[
            pl.BlockSpec((B_BLK, S_BLK, d), lambda s, b: (b, s, 0)),
            pl.BlockSpec((B_BLK, 1, 1, S_BLK), lambda s, b: (b, s, 0, 0)),
            pl.BlockSpec((S_BLK, d), lambda s, b: (s, 0)),
        ],
        out_specs=pl.BlockSpec((B_BLK, S_BLK, d), lambda s, b: (b, s, 0)),
        out_shape=jax.ShapeDtypeStruct((bs, sl, d), x.dtype),
    )(x, mask4, pos_emb)


# speedup vs baseline: 1.0072x; 1.0072x over previous
import jax
import jax.numpy as jnp
from jax.experimental import pallas as pl

S_BLK = 512
B_BLK = 2


def _pe_kernel(x_ref, mask_ref, pe_ref, out_ref):
    m = mask_ref[:, 0, 0, :]
    out_ref[...] = (x_ref[...] + pe_ref[...]) * m[:, :, None]


def kernel(x, mask, pos_emb):
    bs, sl, d = x.shape
    grid = (sl // S_BLK, bs // B_BLK)
    mask4 = mask.reshape(bs, sl // S_BLK, 1, S_BLK)
    return pl.pallas_call(
        _pe_kernel,
        grid=grid,
        in_specs=[
            pl.BlockSpec((B_BLK, S_BLK, d), lambda s, b: (b, s, 0)),
            pl.BlockSpec((B_BLK, 1, 1, S_BLK), lambda s, b: (b, s, 0, 0)),
            pl.BlockSpec((S_BLK, d), lambda s, b: (s, 0)),
        ],
        out_specs=pl.BlockSpec((B_BLK, S_BLK, d), lambda s, b: (b, s, 0)),
        out_shape=jax.ShapeDtypeStruct((bs, sl, d), x.dtype),
    )(x, mask4, pos_emb)
